# baseline (speedup 1.0000x reference)
import jax
import jax.numpy as jnp
from jax.experimental import pallas as pl
from jax.experimental.pallas import tpu as pltpu

_N = 10000
_H = 2048
_D = 128

_HB = 256
_NB = 1000


def _mm_rows_kernel(a_ref, b_ref, o_ref):
    a = a_ref[...].astype(jnp.bfloat16)
    b = b_ref[...].astype(jnp.bfloat16)
    o_ref[...] = jnp.dot(a, b, preferred_element_type=jnp.float32)


def _layer_out_kernel(src_ref, t_ref, x_ref, o_ref):
    s = src_ref[...].astype(jnp.bfloat16)
    t = t_ref[...].astype(jnp.bfloat16)
    y = jnp.dot(s, t, preferred_element_type=jnp.float32)
    o_ref[...] = jnp.maximum(y, 0.0) + x_ref[...]


def _final_kernel(w_ref, src_ref, t_ref, x0_ref, x1_ref, o_ref):
    s = src_ref[...].astype(jnp.bfloat16)
    t = t_ref[...].astype(jnp.bfloat16)
    y = jnp.dot(s, t, preferred_element_type=jnp.float32)
    x1 = x1_ref[...]
    x2 = jnp.maximum(y, 0.0) + x1
    o_ref[...] = w_ref[0] * x0_ref[...] + w_ref[1] * x1 + w_ref[2] * x2


def _mm_rows(a, b, hb):
    r, k = a.shape
    d = b.shape[1]
    return pl.pallas_call(
        _mm_rows_kernel,
        grid=(r // hb,),
        in_specs=[
            pl.BlockSpec((hb, k), lambda i: (i, 0)),
            pl.BlockSpec((k, d), lambda i: (0, 0)),
        ],
        out_specs=pl.BlockSpec((hb, d), lambda i: (i, 0)),
        out_shape=jax.ShapeDtypeStruct((r, d), jnp.float32),
    )(a, b)


def kernel(pois_embs, HG_poi_src, HG_poi_tar, layer_attention):
    w = jax.nn.softmax(layer_attention, axis=0)

    t1 = _mm_rows(HG_poi_tar, pois_embs, _HB)
    x1 = pl.pallas_call(
        _layer_out_kernel,
        grid=(_N // _NB,),
        in_specs=[
            pl.BlockSpec((_NB, _H), lambda i: (i, 0)),
            pl.BlockSpec((_H, _D), lambda i: (0, 0)),
            pl.BlockSpec((_NB, _D), lambda i: (i, 0)),
        ],
        out_specs=pl.BlockSpec((_NB, _D), lambda i: (i, 0)),
        out_shape=jax.ShapeDtypeStruct((_N, _D), jnp.float32),
    )(HG_poi_src, t1, pois_embs)

    t2 = _mm_rows(HG_poi_tar, x1, _HB)
    out = pl.pallas_call(
        _final_kernel,
        grid=(_N // _NB,),
        in_specs=[
            pl.BlockSpec(memory_space=pltpu.SMEM),
            pl.BlockSpec((_NB, _H), lambda i: (i, 0)),
            pl.BlockSpec((_H, _D), lambda i: (0, 0)),
            pl.BlockSpec((_NB, _D), lambda i: (i, 0)),
            pl.BlockSpec((_NB, _D), lambda i: (i, 0)),
        ],
        out_specs=pl.BlockSpec((_NB, _D), lambda i: (i, 0)),
        out_shape=jax.ShapeDtypeStruct((_N, _D), jnp.float32),
    )(w, HG_poi_src, t2, pois_embs, x1)
    return out
